# baseline (speedup 1.0000x reference)
import jax
import jax.numpy as jnp
from jax.experimental import pallas as pl

D = 768
E = 8
K = 2
FF = 3072
S = 2048

T = 256
NT = S // T


def _router_body(x_ref, wg_ref, coef_ref, aux_ref):
    x = x_ref[...]
    wg = wg_ref[...]
    logits = jnp.dot(x, wg, preferred_element_type=jnp.float32)
    m = jnp.max(logits, axis=1, keepdims=True)
    p = jnp.exp(logits - m)
    p = p / jnp.sum(p, axis=1, keepdims=True)
    lane = jax.lax.broadcasted_iota(jnp.int32, (S, E), 1)
    m1 = jnp.max(p, axis=1, keepdims=True)
    i1 = jnp.min(jnp.where(p == m1, lane, E), axis=1, keepdims=True)
    p2 = jnp.where(lane == i1, -jnp.inf, p)
    m2 = jnp.max(p2, axis=1, keepdims=True)
    i2 = jnp.min(jnp.where(p2 == m2, lane, E), axis=1, keepdims=True)
    tot = m1 + m2
    coef = (jnp.where(lane == i1, m1 / tot, 0.0)
            + jnp.where(lane == i2, m2 / tot, 0.0))
    coef_ref[...] = coef
    me = jnp.mean(p, axis=0, keepdims=True)
    cnt = jnp.sum((lane == i1).astype(jnp.float32)
                  + (lane == i2).astype(jnp.float32), axis=0, keepdims=True)
    ce = cnt / (S * K)
    aux_ref[...] = E * jnp.sum(me * ce, keepdims=True)


FT = 1536
NF = FF // FT


def _moe_body(x_ref, coef_ref, w1_ref, w3_ref, w2_ref, out_ref):
    e = pl.program_id(1)
    f = pl.program_id(2)
    x = x_ref[...]
    a = jnp.dot(x, w1_ref[0], preferred_element_type=jnp.float32)
    b = jnp.dot(x, w3_ref[0], preferred_element_type=jnp.float32)
    h = (a / (1.0 + jnp.exp(-a))) * b
    ye = jnp.dot(h, w2_ref[0], preferred_element_type=jnp.float32)
    lane = jax.lax.broadcasted_iota(jnp.int32, (T, E), 1)
    c = jnp.sum(jnp.where(lane == e, coef_ref[...], 0.0), axis=1, keepdims=True)
    contrib = c * ye

    @pl.when((e == 0) & (f == 0))
    def _():
        out_ref[...] = contrib

    @pl.when((e != 0) | (f != 0))
    def _():
        out_ref[...] += contrib


def kernel(x, Wg, W1, W3, W2):
    xf = x.reshape(S, D)
    coef, aux = pl.pallas_call(
        _router_body,
        out_shape=[jax.ShapeDtypeStruct((S, E), jnp.float32),
                   jax.ShapeDtypeStruct((1, 1), jnp.float32)],
    )(xf, Wg)
    y = pl.pallas_call(
        _moe_body,
        grid=(NT, E, NF),
        in_specs=[
            pl.BlockSpec((T, D), lambda t, e, f: (t, 0)),
            pl.BlockSpec((T, E), lambda t, e, f: (t, 0)),
            pl.BlockSpec((1, D, FT), lambda t, e, f: (e, 0, f)),
            pl.BlockSpec((1, D, FT), lambda t, e, f: (e, 0, f)),
            pl.BlockSpec((1, FT, D), lambda t, e, f: (e, f, 0)),
        ],
        out_specs=pl.BlockSpec((T, D), lambda t, e, f: (t, 0)),
        out_shape=jax.ShapeDtypeStruct((S, D), jnp.float32),
    )(xf, coef, W1, W3, W2)
    return y.reshape(1, S, D), aux.reshape(())


# speedup vs baseline: 2.1026x; 2.1026x over previous
import functools

import jax
import jax.numpy as jnp
from jax import lax
from jax.experimental import pallas as pl
from jax.experimental.pallas import tpu as pltpu
from jax.experimental.pallas import tpu_sc as plsc

D = 768
E = 8
K = 2
FF = 3072
S = 2048
P = K * S

T2 = 128
NT2 = P // T2
G = NT2 + E
FT = 1536
NF = FF // FT

NC = 2
NS = 16
NW = NC * NS
CH = P // NW


def _router_body(x_ref, wg_ref, w2_ref, dest_ref, dhi_ref, pe_ref, pt_ref,
                 vl_ref, lo_ref, hi_ref, aux_ref):
    x = x_ref[...]
    wg = wg_ref[...]
    logits = jnp.dot(x, wg, preferred_element_type=jnp.float32)
    m = jnp.max(logits, axis=1, keepdims=True)
    p = jnp.exp(logits - m)
    p = p / jnp.sum(p, axis=1, keepdims=True)
    lane = jax.lax.broadcasted_iota(jnp.int32, (S, E), 1)
    m1 = jnp.max(p, axis=1, keepdims=True)
    i1 = jnp.min(jnp.where(p == m1, lane, E), axis=1, keepdims=True)
    pm = jnp.where(lane == i1, -jnp.inf, p)
    m2 = jnp.max(pm, axis=1, keepdims=True)
    i2 = jnp.min(jnp.where(pm == m2, lane, E), axis=1, keepdims=True)
    tot = m1 + m2
    w2_ref[...] = jnp.concatenate([m1 / tot, m2 / tot], axis=1)

    oh1 = (lane == i1).astype(jnp.float32)
    oh2 = (lane == i2).astype(jnp.float32)

    me = jnp.mean(p, axis=0, keepdims=True)
    cnt1 = jnp.sum(oh1, axis=0, keepdims=True)
    cnt2 = jnp.sum(oh2, axis=0, keepdims=True)
    cnt = cnt1 + cnt2
    aux_ref[...] = E * jnp.sum(me * (cnt / (S * K)), keepdims=True)

    r_i = jax.lax.broadcasted_iota(jnp.int32, (S, S), 0)
    c_i = jax.lax.broadcasted_iota(jnp.int32, (S, S), 1)
    lt = (c_i < r_i).astype(jnp.bfloat16)
    ohb = jnp.concatenate([oh1, oh2], axis=1).astype(jnp.bfloat16)
    pp = jnp.dot(lt, ohb, preferred_element_type=jnp.float32)
    p1 = pp[:, :E]
    p2 = pp[:, E:]

    e_r = jax.lax.broadcasted_iota(jnp.int32, (E, E), 0)
    e_c = jax.lax.broadcasted_iota(jnp.int32, (E, E), 1)
    u_strict = (e_r < e_c).astype(jnp.float32)
    u_incl = (e_r <= e_c).astype(jnp.float32)
    lo = jnp.dot(cnt, u_strict, preferred_element_type=jnp.float32,
                 precision=jax.lax.Precision.HIGHEST)
    hi = lo + cnt
    lo_ref[...] = lo.astype(jnp.int32)
    hi_ref[...] = hi.astype(jnp.int32)

    d1 = jnp.sum(oh1 * (lo + p1), axis=1, keepdims=True)
    d2 = jnp.sum(oh2 * (lo + cnt1 + p2), axis=1, keepdims=True)
    dest = jnp.concatenate([d1, d2], axis=0).astype(jnp.int32)
    dest_ref[...] = dest
    dhi_ref[...] = dest + P

    lo_i = lo.astype(jnp.int32)
    hi_i = hi.astype(jnp.int32)
    ft = jax.lax.shift_right_logical(lo_i, 7)
    ltile = jax.lax.shift_right_logical(hi_i - 1, 7)
    ntl = jnp.where(hi_i > lo_i, ltile - ft + 1, 0)
    ssi = jnp.dot(ntl.astype(jnp.float32), u_incl,
                  preferred_element_type=jnp.float32,
                  precision=jax.lax.Precision.HIGHEST).astype(jnp.int32)
    ssx = ssi - ntl
    total = ssi[:, E - 1:E]

    g8 = jax.lax.broadcasted_iota(jnp.int32, (G, E), 0)
    e_g = jnp.sum((ssi <= g8).astype(jnp.int32), axis=1, keepdims=True)
    e_g = jnp.minimum(e_g, E - 1)
    ohg = (jax.lax.broadcasted_iota(jnp.int32, (G, E), 1) == e_g)
    ohg = ohg.astype(jnp.int32)
    ft_g = jnp.sum(ohg * ft, axis=1, keepdims=True)
    ssx_g = jnp.sum(ohg * ssx, axis=1, keepdims=True)
    gcol = jax.lax.broadcasted_iota(jnp.int32, (G, 1), 0)
    t_g = ft_g + (gcol - ssx_g)
    t_g = jnp.clip(t_g, 0, NT2 - 1)
    pe_ref[...] = e_g
    pt_ref[...] = t_g
    vl_ref[...] = (gcol < total).astype(jnp.int32)


@functools.lru_cache(maxsize=1)
def _sc_kernels():
    mesh = plsc.VectorSubcoreMesh(core_axis_name="c", subcore_axis_name="s")

    @functools.partial(
        pl.kernel, mesh=mesh,
        out_type=jax.ShapeDtypeStruct((P, D), jnp.float32),
        scratch_types=[pltpu.VMEM((CH,), jnp.int32),
                       pltpu.VMEM((CH, D), jnp.float32),
                       pltpu.SemaphoreType.DMA],
    )
    def sc_dispatch(x_hbm, dest_hbm, xs_hbm, idx_v, rows_v, sem):
        wid = lax.axis_index("s") * NC + lax.axis_index("c")
        base = pl.multiple_of(wid * CH, CH)
        tstart = pl.multiple_of(jnp.bitwise_and(base, S - 1), CH)
        pltpu.sync_copy(x_hbm.at[pl.ds(tstart, CH)], rows_v)
        pltpu.sync_copy(dest_hbm.at[pl.ds(base, CH)], idx_v)
        pltpu.async_copy(rows_v, xs_hbm.at[idx_v], sem).wait()

    @functools.partial(
        pl.kernel, mesh=mesh,
        out_type=jax.ShapeDtypeStruct((NF * P, D), jnp.float32),
        scratch_types=[pltpu.VMEM((CH,), jnp.int32),
                       pltpu.VMEM((CH, D), jnp.float32),
                       pltpu.SemaphoreType.DMA],
    )
    def sc_gather_back(ys_hbm, dest_hbm, dhi_hbm, yp_hbm, idx_v, rows_v, sem):
        wid = lax.axis_index("s") * NC + lax.axis_index("c")
        base = pl.multiple_of(wid * CH, CH)
        pltpu.sync_copy(dest_hbm.at[pl.ds(base, CH)], idx_v)
        pltpu.async_copy(ys_hbm.at[idx_v], rows_v, sem).wait()
        pltpu.sync_copy(rows_v, yp_hbm.at[pl.ds(base, CH)])
        pltpu.sync_copy(dhi_hbm.at[pl.ds(base, CH)], idx_v)
        pltpu.async_copy(ys_hbm.at[idx_v], rows_v, sem).wait()
        pltpu.sync_copy(rows_v, yp_hbm.at[pl.ds(P + base, CH)])

    return sc_dispatch, sc_gather_back


def _sc_dispatch(xf, dest1):
    return _sc_kernels()[0](xf, dest1)


def _sc_gather_back(ys_flat, dest1, dhi1):
    return _sc_kernels()[1](ys_flat, dest1, dhi1)


def _gemm_body(pe_ref, pt_ref, vl_ref, fr_ref, lo_ref, hi_ref,
               xs_ref, w1_ref, w3_ref, w2_ref, out_ref):
    g = pl.program_id(1)
    e = pe_ref[g]
    t = pt_ref[g]

    @pl.when(vl_ref[g] == 1)
    def _():
        x = xs_ref[...]
        a = jnp.dot(x, w1_ref[0], preferred_element_type=jnp.float32)
        b = jnp.dot(x, w3_ref[0], preferred_element_type=jnp.float32)
        h = (a / (1.0 + jnp.exp(-a))) * b
        ye = jnp.dot(h, w2_ref[0], preferred_element_type=jnp.float32)
        qrow = t * T2 + jax.lax.broadcasted_iota(jnp.int32, (T2, 1), 0)
        msk = (qrow >= lo_ref[e]) & (qrow < hi_ref[e])
        contrib = jnp.where(msk, ye, 0.0)

        @pl.when(fr_ref[g] == 1)
        def _():
            out_ref[0] = contrib

        @pl.when(fr_ref[g] == 0)
        def _():
            out_ref[0] += contrib


def _combine_body(w_ref, a0_ref, a1_ref, b0_ref, b1_ref, out_ref):
    lane = jax.lax.broadcasted_iota(jnp.int32, (512, K), 1)
    wv = w_ref[...]
    c0 = jnp.sum(jnp.where(lane == 0, wv, 0.0), axis=1, keepdims=True)
    c1 = jnp.sum(jnp.where(lane == 1, wv, 0.0), axis=1, keepdims=True)
    out_ref[...] = (c0 * (a0_ref[0] + b0_ref[0])
                    + c1 * (a1_ref[0] + b1_ref[0]))


def _router_call(xf, Wg):
    return pl.pallas_call(
        _router_body,
        out_shape=[
            jax.ShapeDtypeStruct((S, K), jnp.float32),
            jax.ShapeDtypeStruct((P, 1), jnp.int32),
            jax.ShapeDtypeStruct((P, 1), jnp.int32),
            jax.ShapeDtypeStruct((G, 1), jnp.int32),
            jax.ShapeDtypeStruct((G, 1), jnp.int32),
            jax.ShapeDtypeStruct((G, 1), jnp.int32),
            jax.ShapeDtypeStruct((1, E), jnp.int32),
            jax.ShapeDtypeStruct((1, E), jnp.int32),
            jax.ShapeDtypeStruct((1, 1), jnp.float32),
        ],
    )(xf, Wg)


def kernel(x, Wg, W1, W3, W2):
    xf = x.reshape(S, D)
    w2, dest, dhi, pe, pt, vl, lo, hi, aux = _router_call(xf, Wg)

    dest1 = dest.reshape(P)
    dhi1 = dhi.reshape(P)
    pe1 = pe.reshape(G)
    pt1 = pt.reshape(G)
    vl1 = vl.reshape(G)
    lo1 = lo.reshape(E)
    hi1 = hi.reshape(E)
    fr1 = jnp.concatenate(
        [jnp.ones((1,), jnp.int32), (pt1[1:] != pt1[:-1]).astype(jnp.int32)])

    xs = _sc_dispatch(xf, dest1)

    ysp = _gemm_call(pe1, pt1, vl1, fr1, lo1, hi1, xs, W1, W3, W2)

    yp = _sc_gather_back(ysp.reshape(NF * P, D), dest1, dhi1)
    yp2 = yp.reshape(NF, P, D)

    y = _combine_call(w2, yp2)

    return y.reshape(1, S, D), aux.reshape(())


def _gemm_call(pe1, pt1, vl1, fr1, lo1, hi1, xs, W1, W3, W2):
    return pl.pallas_call(
        _gemm_body,
        grid_spec=pltpu.PrefetchScalarGridSpec(
            num_scalar_prefetch=6,
            grid=(NF, G),
            in_specs=[
                pl.BlockSpec((T2, D),
                             lambda f, g, pe, pt, vl, fr, lo, hi: (pt[g], 0)),
                pl.BlockSpec((1, D, FT),
                             lambda f, g, pe, pt, vl, fr, lo, hi: (pe[g], 0, f)),
                pl.BlockSpec((1, D, FT),
                             lambda f, g, pe, pt, vl, fr, lo, hi: (pe[g], 0, f)),
                pl.BlockSpec((1, FT, D),
                             lambda f, g, pe, pt, vl, fr, lo, hi: (pe[g], f, 0)),
            ],
            out_specs=pl.BlockSpec(
                (1, T2, D),
                lambda f, g, pe, pt, vl, fr, lo, hi: (f, pt[g], 0)),
        ),
        out_shape=jax.ShapeDtypeStruct((NF, P, D), jnp.float32),
    )(pe1, pt1, vl1, fr1, lo1, hi1, xs, W1, W3, W2)


def _combine_call(w2, yp2):
    NB = S // 512
    return pl.pallas_call(
        _combine_body,
        grid=(NB,),
        in_specs=[
            pl.BlockSpec((512, K), lambda t: (t, 0)),
            pl.BlockSpec((1, 512, D), lambda t: (0, t, 0)),
            pl.BlockSpec((1, 512, D), lambda t: (0, NB + t, 0)),
            pl.BlockSpec((1, 512, D), lambda t: (1, t, 0)),
            pl.BlockSpec((1, 512, D), lambda t: (1, NB + t, 0)),
        ],
        out_specs=pl.BlockSpec((512, D), lambda t: (t, 0)),
        out_shape=jax.ShapeDtypeStruct((S, D), jnp.float32),
    )(w2, yp2, yp2, yp2, yp2)


# speedup vs baseline: 2.2504x; 1.0703x over previous
import functools

import jax
import jax.numpy as jnp
from jax import lax
from jax.experimental import pallas as pl
from jax.experimental.pallas import tpu as pltpu
from jax.experimental.pallas import tpu_sc as plsc

D = 768
E = 8
K = 2
FF = 3072
S = 2048
P = K * S

T2 = 256
NT2 = P // T2
G = NT2 + E
FT = 1536
NF = FF // FT

NC = 2
NS = 16
NW = NC * NS
CH = P // NW


def _router_body(x_ref, wg_ref, w2_ref, dest_ref, dhi_ref, pe_ref, pt_ref,
                 vl_ref, lo_ref, hi_ref, aux_ref):
    x = x_ref[...]
    wg = wg_ref[...]
    logits = jnp.dot(x, wg, preferred_element_type=jnp.float32)
    m = jnp.max(logits, axis=1, keepdims=True)
    p = jnp.exp(logits - m)
    p = p / jnp.sum(p, axis=1, keepdims=True)
    lane = jax.lax.broadcasted_iota(jnp.int32, (S, E), 1)
    m1 = jnp.max(p, axis=1, keepdims=True)
    i1 = jnp.min(jnp.where(p == m1, lane, E), axis=1, keepdims=True)
    pm = jnp.where(lane == i1, -jnp.inf, p)
    m2 = jnp.max(pm, axis=1, keepdims=True)
    i2 = jnp.min(jnp.where(pm == m2, lane, E), axis=1, keepdims=True)
    tot = m1 + m2
    w2_ref[...] = jnp.concatenate([m1 / tot, m2 / tot], axis=1)

    oh1 = (lane == i1).astype(jnp.float32)
    oh2 = (lane == i2).astype(jnp.float32)

    me = jnp.mean(p, axis=0, keepdims=True)
    cnt1 = jnp.sum(oh1, axis=0, keepdims=True)
    cnt2 = jnp.sum(oh2, axis=0, keepdims=True)
    cnt = cnt1 + cnt2
    aux_ref[...] = E * jnp.sum(me * (cnt / (S * K)), keepdims=True)

    r_i = jax.lax.broadcasted_iota(jnp.int32, (S, S), 0)
    c_i = jax.lax.broadcasted_iota(jnp.int32, (S, S), 1)
    lt = (c_i < r_i).astype(jnp.bfloat16)
    ohb = jnp.concatenate([oh1, oh2], axis=1).astype(jnp.bfloat16)
    pp = jnp.dot(lt, ohb, preferred_element_type=jnp.float32)
    p1 = pp[:, :E]
    p2 = pp[:, E:]

    e_r = jax.lax.broadcasted_iota(jnp.int32, (E, E), 0)
    e_c = jax.lax.broadcasted_iota(jnp.int32, (E, E), 1)
    u_strict = (e_r < e_c).astype(jnp.float32)
    u_incl = (e_r <= e_c).astype(jnp.float32)
    lo = jnp.dot(cnt, u_strict, preferred_element_type=jnp.float32,
                 precision=jax.lax.Precision.HIGHEST)
    hi = lo + cnt
    lo_ref[...] = lo.astype(jnp.int32)
    hi_ref[...] = hi.astype(jnp.int32)

    d1 = jnp.sum(oh1 * (lo + p1), axis=1, keepdims=True)
    d2 = jnp.sum(oh2 * (lo + cnt1 + p2), axis=1, keepdims=True)
    dest = jnp.concatenate([d1, d2], axis=0).astype(jnp.int32)
    dest_ref[...] = dest
    dhi_ref[...] = dest + P

    lo_i = lo.astype(jnp.int32)
    hi_i = hi.astype(jnp.int32)
    ft = jax.lax.shift_right_logical(lo_i, 8)
    ltile = jax.lax.shift_right_logical(hi_i - 1, 8)
    ntl = jnp.where(hi_i > lo_i, ltile - ft + 1, 0)
    ssi = jnp.dot(ntl.astype(jnp.float32), u_incl,
                  preferred_element_type=jnp.float32,
                  precision=jax.lax.Precision.HIGHEST).astype(jnp.int32)
    ssx = ssi - ntl
    total = ssi[:, E - 1:E]

    g8 = jax.lax.broadcasted_iota(jnp.int32, (G, E), 0)
    e_g = jnp.sum((ssi <= g8).astype(jnp.int32), axis=1, keepdims=True)
    e_g = jnp.minimum(e_g, E - 1)
    ohg = (jax.lax.broadcasted_iota(jnp.int32, (G, E), 1) == e_g)
    ohg = ohg.astype(jnp.int32)
    ft_g = jnp.sum(ohg * ft, axis=1, keepdims=True)
    ssx_g = jnp.sum(ohg * ssx, axis=1, keepdims=True)
    gcol = jax.lax.broadcasted_iota(jnp.int32, (G, 1), 0)
    t_g = ft_g + (gcol - ssx_g)
    t_g = jnp.clip(t_g, 0, NT2 - 1)
    pe_ref[...] = e_g
    pt_ref[...] = t_g
    vl_ref[...] = (gcol < total).astype(jnp.int32)


@functools.lru_cache(maxsize=1)
def _sc_kernels():
    mesh = plsc.VectorSubcoreMesh(core_axis_name="c", subcore_axis_name="s")

    @functools.partial(
        pl.kernel, mesh=mesh,
        out_type=jax.ShapeDtypeStruct((P, D), jnp.float32),
        scratch_types=[pltpu.VMEM((CH,), jnp.int32),
                       pltpu.VMEM((CH, D), jnp.float32),
                       pltpu.SemaphoreType.DMA],
    )
    def sc_dispatch(x_hbm, dest_hbm, xs_hbm, idx_v, rows_v, sem):
        wid = lax.axis_index("s") * NC + lax.axis_index("c")
        base = pl.multiple_of(wid * CH, CH)
        tstart = pl.multiple_of(jnp.bitwise_and(base, S - 1), CH)
        pltpu.sync_copy(x_hbm.at[pl.ds(tstart, CH)], rows_v)
        pltpu.sync_copy(dest_hbm.at[pl.ds(base, CH)], idx_v)
        pltpu.async_copy(rows_v, xs_hbm.at[idx_v], sem).wait()

    @functools.partial(
        pl.kernel, mesh=mesh,
        out_type=jax.ShapeDtypeStruct((NF * P, D), jnp.float32),
        scratch_types=[pltpu.VMEM((CH,), jnp.int32),
                       pltpu.VMEM((CH, D), jnp.float32),
                       pltpu.SemaphoreType.DMA],
    )
    def sc_gather_back(ys_hbm, dest_hbm, dhi_hbm, yp_hbm, idx_v, rows_v, sem):
        wid = lax.axis_index("s") * NC + lax.axis_index("c")
        base = pl.multiple_of(wid * CH, CH)
        pltpu.sync_copy(dest_hbm.at[pl.ds(base, CH)], idx_v)
        pltpu.async_copy(ys_hbm.at[idx_v], rows_v, sem).wait()
        pltpu.sync_copy(rows_v, yp_hbm.at[pl.ds(base, CH)])
        pltpu.sync_copy(dhi_hbm.at[pl.ds(base, CH)], idx_v)
        pltpu.async_copy(ys_hbm.at[idx_v], rows_v, sem).wait()
        pltpu.sync_copy(rows_v, yp_hbm.at[pl.ds(P + base, CH)])

    return sc_dispatch, sc_gather_back


def _sc_dispatch(xf, dest1):
    return _sc_kernels()[0](xf, dest1)


def _sc_gather_back(ys_flat, dest1, dhi1):
    return _sc_kernels()[1](ys_flat, dest1, dhi1)


def _gemm_body(pe_ref, pt_ref, vl_ref, fr_ref, lo_ref, hi_ref,
               xs_ref, w1_ref, w3_ref, w2_ref, out_ref):
    g = pl.program_id(1)
    e = pe_ref[g]
    t = pt_ref[g]

    @pl.when(vl_ref[g] == 1)
    def _():
        x = xs_ref[...]
        a = jnp.dot(x, w1_ref[0], preferred_element_type=jnp.float32)
        b = jnp.dot(x, w3_ref[0], preferred_element_type=jnp.float32)
        h = (a / (1.0 + jnp.exp(-a))) * b
        ye = jnp.dot(h, w2_ref[0], preferred_element_type=jnp.float32)
        qrow = t * T2 + jax.lax.broadcasted_iota(jnp.int32, (T2, 1), 0)
        msk = (qrow >= lo_ref[e]) & (qrow < hi_ref[e])
        contrib = jnp.where(msk, ye, 0.0)

        @pl.when(fr_ref[g] == 1)
        def _():
            out_ref[0] = contrib

        @pl.when(fr_ref[g] == 0)
        def _():
            out_ref[0] += contrib


def _combine_body(w_ref, a0_ref, a1_ref, b0_ref, b1_ref, out_ref):
    lane = jax.lax.broadcasted_iota(jnp.int32, (512, K), 1)
    wv = w_ref[...]
    c0 = jnp.sum(jnp.where(lane == 0, wv, 0.0), axis=1, keepdims=True)
    c1 = jnp.sum(jnp.where(lane == 1, wv, 0.0), axis=1, keepdims=True)
    out_ref[...] = (c0 * (a0_ref[0] + b0_ref[0])
                    + c1 * (a1_ref[0] + b1_ref[0]))


def _router_call(xf, Wg):
    return pl.pallas_call(
        _router_body,
        out_shape=[
            jax.ShapeDtypeStruct((S, K), jnp.float32),
            jax.ShapeDtypeStruct((P, 1), jnp.int32),
            jax.ShapeDtypeStruct((P, 1), jnp.int32),
            jax.ShapeDtypeStruct((G, 1), jnp.int32),
            jax.ShapeDtypeStruct((G, 1), jnp.int32),
            jax.ShapeDtypeStruct((G, 1), jnp.int32),
            jax.ShapeDtypeStruct((1, E), jnp.int32),
            jax.ShapeDtypeStruct((1, E), jnp.int32),
            jax.ShapeDtypeStruct((1, 1), jnp.float32),
        ],
    )(xf, Wg)


def kernel(x, Wg, W1, W3, W2):
    xf = x.reshape(S, D)
    w2, dest, dhi, pe, pt, vl, lo, hi, aux = _router_call(xf, Wg)

    dest1 = dest.reshape(P)
    dhi1 = dhi.reshape(P)
    pe1 = pe.reshape(G)
    pt1 = pt.reshape(G)
    vl1 = vl.reshape(G)
    lo1 = lo.reshape(E)
    hi1 = hi.reshape(E)
    fr1 = jnp.concatenate(
        [jnp.ones((1,), jnp.int32), (pt1[1:] != pt1[:-1]).astype(jnp.int32)])

    xs = _sc_dispatch(xf, dest1)

    ysp = _gemm_call(pe1, pt1, vl1, fr1, lo1, hi1, xs, W1, W3, W2)

    yp = _sc_gather_back(ysp.reshape(NF * P, D), dest1, dhi1)
    yp2 = yp.reshape(NF, P, D)

    y = _combine_call(w2, yp2)

    return y.reshape(1, S, D), aux.reshape(())


def _gemm_call(pe1, pt1, vl1, fr1, lo1, hi1, xs, W1, W3, W2):
    return pl.pallas_call(
        _gemm_body,
        grid_spec=pltpu.PrefetchScalarGridSpec(
            num_scalar_prefetch=6,
            grid=(NF, G),
            in_specs=[
                pl.BlockSpec((T2, D),
                             lambda f, g, pe, pt, vl, fr, lo, hi: (pt[g], 0)),
                pl.BlockSpec((1, D, FT),
                             lambda f, g, pe, pt, vl, fr, lo, hi: (pe[g], 0, f)),
                pl.BlockSpec((1, D, FT),
                             lambda f, g, pe, pt, vl, fr, lo, hi: (pe[g], 0, f)),
                pl.BlockSpec((1, FT, D),
                             lambda f, g, pe, pt, vl, fr, lo, hi: (pe[g], f, 0)),
            ],
            out_specs=pl.BlockSpec(
                (1, T2, D),
                lambda f, g, pe, pt, vl, fr, lo, hi: (f, pt[g], 0)),
        ),
        out_shape=jax.ShapeDtypeStruct((NF, P, D), jnp.float32),
    )(pe1, pt1, vl1, fr1, lo1, hi1, xs, W1, W3, W2)


def _combine_call(w2, yp2):
    NB = S // 512
    return pl.pallas_call(
        _combine_body,
        grid=(NB,),
        in_specs=[
            pl.BlockSpec((512, K), lambda t: (t, 0)),
            pl.BlockSpec((1, 512, D), lambda t: (0, t, 0)),
            pl.BlockSpec((1, 512, D), lambda t: (0, NB + t, 0)),
            pl.BlockSpec((1, 512, D), lambda t: (1, t, 0)),
            pl.BlockSpec((1, 512, D), lambda t: (1, NB + t, 0)),
        ],
        out_specs=pl.BlockSpec((512, D), lambda t: (t, 0)),
        out_shape=jax.ShapeDtypeStruct((S, D), jnp.float32),
    )(w2, yp2, yp2, yp2, yp2)
